# baseline (speedup 1.0000x reference)
import functools

import jax
import jax.numpy as jnp
from jax import lax
from jax.experimental import pallas as pl
from jax.experimental.pallas import tpu as pltpu
from jax.experimental.pallas import tpu_sc as plsc

B = 4096
L = 50
V = 100000
D = 128
H = 256
C = 2

NC = 2
NS = 16
NW = NC * NS
BAGS_PER_W = B // NW
CHUNK_BAGS = 2
CHUNK_IDX = CHUNK_BAGS * L
CHUNKS_PER_W = BAGS_PER_W // CHUNK_BAGS
G = D // 16


IDX_PER_W = BAGS_PER_W * L
NBUF = 4
GATHER_ROWS = CHUNK_IDX + 4


def _sc_bag_sums_body(table_hbm, tags_hbm, out_hbm, idx_v, rows0, rows1,
                      rows2, rows3, out_v, sem0, sem1, sem2, sem3):
    cid = lax.axis_index("c")
    sid = lax.axis_index("s")
    wid = sid * NC + cid
    bufs = (rows0, rows1, rows2, rows3)
    sems = (sem0, sem1, sem2, sem3)

    pltpu.sync_copy(tags_hbm.at[pl.ds(wid * IDX_PER_W, IDX_PER_W)], idx_v)

    def idx_slice(ci, parity):
        if parity == 0:
            off = pl.multiple_of(ci * CHUNK_IDX, 8)
            return idx_v.at[pl.ds(off, CHUNK_IDX)]
        off = pl.multiple_of(ci * CHUNK_IDX - 4, 8)
        return idx_v.at[pl.ds(off, GATHER_ROWS)]

    for b in range(NBUF):
        dst0 = bufs[b] if b % 2 else bufs[b].at[pl.ds(0, CHUNK_IDX)]
        pltpu.async_copy(table_hbm.at[idx_slice(b, b % 2)], dst0, sems[b])

    def outer(cc, carry):
        for b in range(NBUF):
            ci = NBUF * cc + b
            dst = bufs[b] if b % 2 else bufs[b].at[pl.ds(0, CHUNK_IDX)]
            pltpu.make_async_copy(
                table_hbm.at[idx_slice(ci, b % 2)], dst, sems[b]).wait()
            for b2 in range(CHUNK_BAGS):
                base = 4 * (b % 2) + b2 * L
                zeros = tuple(jnp.zeros((16,), jnp.float32) for _ in range(G))

                @plsc.parallel_loop(base, base + L, unroll=5, carry=zeros)
                def accum(r, acc, _b=b):
                    return tuple(
                        acc[g] + bufs[_b][r, pl.ds(g * 16, 16)]
                        for g in range(G)
                    )

                acc = accum
                row = CHUNK_BAGS * ci + b2
                for g in range(G):
                    out_v[row, pl.ds(g * 16, 16)] = acc[g]
            nci = ci + NBUF

            @pl.when(nci < CHUNKS_PER_W)
            def _():
                pltpu.async_copy(
                    table_hbm.at[idx_slice(nci, b % 2)], dst, sems[b])
        return carry

    lax.fori_loop(0, CHUNKS_PER_W // NBUF, outer, 0)
    pltpu.sync_copy(out_v, out_hbm.at[pl.ds(wid * BAGS_PER_W, BAGS_PER_W)])


@jax.jit
def _sc_bag_sums(emb_weight, tags_vec):
    mesh = plsc.VectorSubcoreMesh(core_axis_name="c", subcore_axis_name="s")
    return pl.kernel(
        _sc_bag_sums_body,
        out_type=jax.ShapeDtypeStruct((B, D), jnp.float32),
        mesh=mesh,
        scratch_types=(
            [pltpu.VMEM((IDX_PER_W,), jnp.int32)]
            + [pltpu.VMEM((GATHER_ROWS, D), jnp.float32)] * NBUF
            + [pltpu.VMEM((BAGS_PER_W, D), jnp.float32)]
            + [pltpu.SemaphoreType.DMA] * NBUF
        ),
    )(emb_weight, tags_vec)


ROWS_BLK = 1024


def _mlp_body(bags_ref, emos_ref, hw_ref, b1_ref, wo_ref, bo_ref, out_ref):
    feats = jnp.maximum(bags_ref[...] * (1.0 / L), 0.0)
    hw = hw_ref[...]
    h = jnp.dot(feats, hw[:, :D].T, preferred_element_type=jnp.float32)
    h = h + jnp.dot(emos_ref[...], hw[:, D:].T,
                    preferred_element_type=jnp.float32)
    h = jnp.maximum(h + b1_ref[...], 0.0)
    out_ref[...] = (
        jnp.dot(h, wo_ref[...].T, preferred_element_type=jnp.float32)
        + bo_ref[...]
    )


@jax.jit
def _mlp(bag_sums, emos, hid_w, b1, wo, bo):
    nblk = B // ROWS_BLK
    return pl.pallas_call(
        _mlp_body,
        out_shape=jax.ShapeDtypeStruct((B, C), jnp.float32),
        grid=(nblk,),
        in_specs=[
            pl.BlockSpec((ROWS_BLK, D), lambda i: (i, 0)),
            pl.BlockSpec((ROWS_BLK, 2), lambda i: (i, 0)),
            pl.BlockSpec((H, D + 2), lambda i: (0, 0)),
            pl.BlockSpec((1, H), lambda i: (0, 0)),
            pl.BlockSpec((C, H), lambda i: (0, 0)),
            pl.BlockSpec((1, C), lambda i: (0, 0)),
        ],
        out_specs=pl.BlockSpec((ROWS_BLK, C), lambda i: (i, 0)),
    )(bag_sums, emos, hid_w, b1, wo, bo)


def kernel(emos, tags_vec, offsets, emb_weight, hid_w, hid_b, out_w, out_b):
    del offsets
    bag_sums = _sc_bag_sums(emb_weight, tags_vec)
    return _mlp(bag_sums, emos, hid_w, hid_b.reshape(1, H), out_w,
                out_b.reshape(1, C))


# speedup vs baseline: 1.0159x; 1.0159x over previous
import functools

import jax
import jax.numpy as jnp
from jax import lax
from jax.experimental import pallas as pl
from jax.experimental.pallas import tpu as pltpu
from jax.experimental.pallas import tpu_sc as plsc

B = 4096
L = 50
V = 100000
D = 128
H = 256
C = 2

NC = 2
NS = 16
NW = NC * NS
BAGS_PER_W = B // NW
CHUNK_BAGS = 2
CHUNK_IDX = CHUNK_BAGS * L
CHUNKS_PER_W = BAGS_PER_W // CHUNK_BAGS
G = D // 16


IDX_PER_W = BAGS_PER_W * L
NBUF = 4
GATHER_ROWS = CHUNK_IDX + 4


def _sc_bag_sums_body(table_hbm, tags_hbm, out_hbm, idx_v, rows0, rows1,
                      rows2, rows3, out_v, sem0, sem1, sem2, sem3, osem):
    cid = lax.axis_index("c")
    sid = lax.axis_index("s")
    wid = sid * NC + cid
    bufs = (rows0, rows1, rows2, rows3)
    sems = (sem0, sem1, sem2, sem3)

    pltpu.sync_copy(tags_hbm.at[pl.ds(wid * IDX_PER_W, IDX_PER_W)], idx_v)

    def idx_slice(ci, parity):
        if parity == 0:
            off = pl.multiple_of(ci * CHUNK_IDX, 8)
            return idx_v.at[pl.ds(off, CHUNK_IDX)]
        off = pl.multiple_of(ci * CHUNK_IDX - 4, 8)
        return idx_v.at[pl.ds(off, GATHER_ROWS)]

    for b in range(NBUF):
        dst0 = bufs[b] if b % 2 else bufs[b].at[pl.ds(0, CHUNK_IDX)]
        pltpu.async_copy(table_hbm.at[idx_slice(b, b % 2)], dst0, sems[b])

    def outer(cc, carry):
        for b in range(NBUF):
            ci = NBUF * cc + b
            dst = bufs[b] if b % 2 else bufs[b].at[pl.ds(0, CHUNK_IDX)]
            pltpu.make_async_copy(
                table_hbm.at[idx_slice(ci, b % 2)], dst, sems[b]).wait()
            for b2 in range(CHUNK_BAGS):
                base = 4 * (b % 2) + b2 * L
                zeros = tuple(jnp.zeros((16,), jnp.float32) for _ in range(G))

                @plsc.parallel_loop(base, base + L, unroll=5, carry=zeros)
                def accum(r, acc, _b=b):
                    return tuple(
                        acc[g] + bufs[_b][r, pl.ds(g * 16, 16)]
                        for g in range(G)
                    )

                acc = accum
                row = CHUNK_BAGS * ci + b2
                for g in range(G):
                    out_v[row, pl.ds(g * 16, 16)] = acc[g]
            orow = CHUNK_BAGS * ci
            pltpu.async_copy(
                out_v.at[pl.ds(orow, CHUNK_BAGS)],
                out_hbm.at[pl.ds(wid * BAGS_PER_W + orow, CHUNK_BAGS)], osem)
            nci = ci + NBUF

            @pl.when(nci < CHUNKS_PER_W)
            def _():
                pltpu.async_copy(
                    table_hbm.at[idx_slice(nci, b % 2)], dst, sems[b])
        return carry

    lax.fori_loop(0, CHUNKS_PER_W // NBUF, outer, 0)
    pltpu.make_async_copy(
        out_hbm.at[pl.ds(wid * BAGS_PER_W, BAGS_PER_W)], out_v, osem).wait()


@jax.jit
def _sc_bag_sums(emb_weight, tags_vec):
    mesh = plsc.VectorSubcoreMesh(core_axis_name="c", subcore_axis_name="s")
    return pl.kernel(
        _sc_bag_sums_body,
        out_type=jax.ShapeDtypeStruct((B, D), jnp.float32),
        mesh=mesh,
        scratch_types=(
            [pltpu.VMEM((IDX_PER_W,), jnp.int32)]
            + [pltpu.VMEM((GATHER_ROWS, D), jnp.float32)] * NBUF
            + [pltpu.VMEM((BAGS_PER_W, D), jnp.float32)]
            + [pltpu.SemaphoreType.DMA] * (NBUF + 1)
        ),
    )(emb_weight, tags_vec)


ROWS_BLK = 2048


def _mlp_body(bags_ref, emos_ref, hw_ref, b1_ref, wo_ref, bo_ref, out_ref):
    feats = jnp.maximum(bags_ref[...] * (1.0 / L), 0.0)
    hw = hw_ref[...]
    h = jnp.dot(feats, hw[:, :D].T, preferred_element_type=jnp.float32)
    h = h + jnp.dot(emos_ref[...], hw[:, D:].T,
                    preferred_element_type=jnp.float32)
    h = jnp.maximum(h + b1_ref[...], 0.0)
    out_ref[...] = (
        jnp.dot(h, wo_ref[...].T, preferred_element_type=jnp.float32)
        + bo_ref[...]
    )


@jax.jit
def _mlp(bag_sums, emos, hid_w, b1, wo, bo):
    nblk = B // ROWS_BLK
    return pl.pallas_call(
        _mlp_body,
        out_shape=jax.ShapeDtypeStruct((B, C), jnp.float32),
        grid=(nblk,),
        in_specs=[
            pl.BlockSpec((ROWS_BLK, D), lambda i: (i, 0)),
            pl.BlockSpec((ROWS_BLK, 2), lambda i: (i, 0)),
            pl.BlockSpec((H, D + 2), lambda i: (0, 0)),
            pl.BlockSpec((1, H), lambda i: (0, 0)),
            pl.BlockSpec((C, H), lambda i: (0, 0)),
            pl.BlockSpec((1, C), lambda i: (0, 0)),
        ],
        out_specs=pl.BlockSpec((ROWS_BLK, C), lambda i: (i, 0)),
    )(bag_sums, emos, hid_w, b1, wo, bo)


def kernel(emos, tags_vec, offsets, emb_weight, hid_w, hid_b, out_w, out_b):
    del offsets
    bag_sums = _sc_bag_sums(emb_weight, tags_vec)
    return _mlp(bag_sums, emos, hid_w, hid_b.reshape(1, H), out_w,
                out_b.reshape(1, C))
